# Initial kernel scaffold; baseline (speedup 1.0000x reference)
import jax
import jax.numpy as jnp
from jax.experimental import pallas as pl


def kernel(x, edge_index, W, b):
    raise NotImplementedError("write your pallas kernel here")


# speedup vs baseline: 1.6978x; 1.6978x over previous
import functools

import jax
import jax.numpy as jnp
from jax import lax
from jax.experimental import pallas as pl
from jax.experimental.pallas import tpu as pltpu
from jax.experimental.pallas import tpu_sc as plsc

D = 128
DC = 2 * D
NC = 2
NS = 16
NW = NC * NS
CH = 80
LANES = 16


def _table_body(x_ref, w_ref, b_ref, c_ref):
    c_ref[...] = (
        jnp.dot(x_ref[...], w_ref[...], preferred_element_type=jnp.float32)
        + b_ref[...]
    )


def _build_table(x, wcat, bcat):
    n = x.shape[0]
    blk = 1000
    grid = n // blk
    return pl.pallas_call(
        _table_body,
        grid=(grid,),
        in_specs=[
            pl.BlockSpec((blk, D), lambda i: (i, 0)),
            pl.BlockSpec((D, DC), lambda i: (0, 0)),
            pl.BlockSpec((1, DC), lambda i: (0, 0)),
        ],
        out_specs=pl.BlockSpec((blk, DC), lambda i: (i, 0)),
        out_shape=jax.ShapeDtypeStruct((n, DC), jnp.float32),
    )(x, wcat, bcat)


def _make_edge_kernel(n_edges):
    per_w = n_edges // NW
    n_chunks = per_w // CH
    mesh = plsc.VectorSubcoreMesh(core_axis_name="c", subcore_axis_name="s")

    @functools.partial(
        pl.kernel,
        mesh=mesh,
        out_type=jax.ShapeDtypeStruct((n_edges, D), jnp.float32),
        scratch_types=[
            pltpu.VMEM((CH,), jnp.int32),
            pltpu.VMEM((CH,), jnp.int32),
            pltpu.VMEM((CH, DC), jnp.float32),
            pltpu.VMEM((CH, DC), jnp.float32),
            pltpu.VMEM((CH, D), jnp.float32),
            pltpu.SemaphoreType.DMA,
        ],
    )
    def edge_kernel(c_hbm, src_hbm, dst_hbm, out_hbm,
                    sidx, didx, srow, drow, orow, sem):
        wid = lax.axis_index("s") * NC + lax.axis_index("c")
        base = wid * per_w

        def chunk_body(ci, carry):
            off = base + ci * CH
            pltpu.sync_copy(src_hbm.at[pl.ds(off, CH)], sidx)
            pltpu.sync_copy(dst_hbm.at[pl.ds(off, CH)], didx)
            cp_s = pltpu.async_copy(c_hbm.at[sidx], srow, sem)
            cp_d = pltpu.async_copy(c_hbm.at[didx], drow, sem)
            cp_s.wait()
            cp_d.wait()

            def row_body(i, rcarry):
                for j in range(D // LANES):
                    lo = j * LANES
                    hi = D + j * LANES
                    s1 = srow[i, pl.ds(lo, LANES)]
                    d2 = drow[i, pl.ds(hi, LANES)]
                    d1 = drow[i, pl.ds(lo, LANES)]
                    s2 = srow[i, pl.ds(hi, LANES)]
                    orow[i, pl.ds(lo, LANES)] = (
                        jnp.maximum(s1 + d2, 0.0) + jnp.maximum(d1 + s2, 0.0)
                    )
                return rcarry

            lax.fori_loop(0, CH, row_body, 0)
            pltpu.sync_copy(orow, out_hbm.at[pl.ds(off, CH)])
            return carry

        lax.fori_loop(0, n_chunks, chunk_body, 0)

    return edge_kernel


def kernel(x, edge_index, W, b):
    n_edges = edge_index.shape[1]
    wcat = jnp.concatenate([W[:D], W[D:]], axis=1)
    bcat = jnp.concatenate([jnp.zeros_like(b), b]).reshape(1, DC)
    c = _build_table(x, wcat, bcat)
    src = edge_index[0].astype(jnp.int32)
    dst = edge_index[1].astype(jnp.int32)
    return _make_edge_kernel(n_edges)(c, src, dst)


# speedup vs baseline: 2.7196x; 1.6018x over previous
import functools

import jax
import jax.numpy as jnp
from jax import lax
from jax.experimental import pallas as pl
from jax.experimental.pallas import tpu as pltpu
from jax.experimental.pallas import tpu_sc as plsc

D = 128
DC = 2 * D
NC = 2
NS = 16
NW = NC * NS
CH = 40
NBUF = 2
LANES = 16


def _table_body(x_ref, w_ref, b_ref, c_ref):
    c_ref[...] = (
        jnp.dot(x_ref[...], w_ref[...], preferred_element_type=jnp.float32)
        + b_ref[...]
    )


def _build_table(x, wcat, bcat):
    n = x.shape[0]
    blk = 1000
    grid = n // blk
    return pl.pallas_call(
        _table_body,
        grid=(grid,),
        in_specs=[
            pl.BlockSpec((blk, D), lambda i: (i, 0)),
            pl.BlockSpec((D, DC), lambda i: (0, 0)),
            pl.BlockSpec((1, DC), lambda i: (0, 0)),
        ],
        out_specs=pl.BlockSpec((blk, DC), lambda i: (i, 0)),
        out_shape=jax.ShapeDtypeStruct((n, DC), jnp.float32),
    )(x, wcat, bcat)


def _make_edge_kernel(n_edges):
    per_w = n_edges // NW
    n_chunks = per_w // CH

    mesh = plsc.VectorSubcoreMesh(core_axis_name="c", subcore_axis_name="s")

    @functools.partial(
        pl.kernel,
        mesh=mesh,
        out_type=jax.ShapeDtypeStruct((n_edges, D), jnp.float32),
        scratch_types=[
            pltpu.VMEM((n_chunks, CH), jnp.int32),
            pltpu.VMEM((n_chunks, CH), jnp.int32),
        ]
        + [pltpu.VMEM((CH, DC), jnp.float32) for _ in range(2 * NBUF)]
        + [pltpu.VMEM((CH, D), jnp.float32) for _ in range(NBUF)]
        + [pltpu.SemaphoreType.DMA for _ in range(2 * NBUF)],
    )
    def edge_kernel(c_hbm, src_hbm, dst_hbm, out_hbm,
                    sidx, didx, srow0, srow1, drow0, drow1, orow0, orow1,
                    gsem0, gsem1, wsem0, wsem1):
        srow = (srow0, srow1)
        drow = (drow0, drow1)
        orow = (orow0, orow1)
        gsem = (gsem0, gsem1)
        wsem = (wsem0, wsem1)
        wid = lax.axis_index("s") * NC + lax.axis_index("c")
        base = wid * per_w

        pltpu.sync_copy(src_hbm.at[wid], sidx)
        pltpu.sync_copy(dst_hbm.at[wid], didx)

        def fire_gathers(b, ci):
            pltpu.async_copy(c_hbm.at[sidx.at[ci]], srow[b], gsem[b])
            pltpu.async_copy(c_hbm.at[didx.at[ci]], drow[b], gsem[b])

        def wait_gathers(b):
            cp = pltpu.make_async_copy(c_hbm.at[sidx.at[0]], srow[b], gsem[b])
            cp.wait()
            cp = pltpu.make_async_copy(c_hbm.at[didx.at[0]], drow[b], gsem[b])
            cp.wait()

        def wait_write(b):
            pltpu.make_async_copy(
                orow[b], out_hbm.at[pl.ds(base, CH)], wsem[b]).wait()

        for b in range(NBUF):
            fire_gathers(b, b)

        def pair_body(ci0_half, carry):
            ci0 = ci0_half * NBUF
            for b in range(NBUF):
                ci = ci0 + b
                wait_gathers(b)

                @pl.when(ci0 >= NBUF - b)
                def _():
                    wait_write(b)

                def row_body(i, rcarry):
                    for j in range(D // LANES):
                        lo = j * LANES
                        hi = D + j * LANES
                        s1 = srow[b][i, pl.ds(lo, LANES)]
                        d2 = drow[b][i, pl.ds(hi, LANES)]
                        d1 = drow[b][i, pl.ds(lo, LANES)]
                        s2 = srow[b][i, pl.ds(hi, LANES)]
                        orow[b][i, pl.ds(lo, LANES)] = (
                            jnp.maximum(s1 + d2, 0.0)
                            + jnp.maximum(d1 + s2, 0.0)
                        )
                    return rcarry

                lax.fori_loop(0, CH, row_body, 0)
                pltpu.async_copy(
                    orow[b], out_hbm.at[pl.ds(base + ci * CH, CH)], wsem[b])

                @pl.when(ci0 + NBUF < n_chunks)
                def _():
                    fire_gathers(b, ci + NBUF)

            return carry

        lax.fori_loop(0, n_chunks // NBUF, pair_body, 0)

        for b in range(NBUF):
            wait_write(b)

    return edge_kernel


def kernel(x, edge_index, W, b):
    n_edges = edge_index.shape[1]
    per_w = n_edges // NW
    n_chunks = per_w // CH
    wcat = jnp.concatenate([W[:D], W[D:]], axis=1)
    bcat = jnp.concatenate([jnp.zeros_like(b), b]).reshape(1, DC)
    c = _build_table(x, wcat, bcat)
    src = edge_index[0].astype(jnp.int32).reshape(NW, n_chunks, CH)
    dst = edge_index[1].astype(jnp.int32).reshape(NW, n_chunks, CH)
    return _make_edge_kernel(n_edges)(c, src, dst)


# speedup vs baseline: 4.9022x; 1.8026x over previous
import functools

import jax
import jax.numpy as jnp
from jax import lax
from jax.experimental import pallas as pl
from jax.experimental.pallas import tpu as pltpu
from jax.experimental.pallas import tpu_sc as plsc

D = 128
DC = 2 * D
NC = 2
NS = 16
NW = NC * NS
CH = 40
NBUF = 2
LANES = 16


def _table_body(x_ref, w_ref, b_ref, c_ref):
    c_ref[...] = (
        jnp.dot(x_ref[...], w_ref[...], preferred_element_type=jnp.float32)
        + b_ref[...]
    ).astype(jnp.bfloat16)


def _build_table(x, wcat, bcat):
    n = x.shape[0]
    blk = 1000
    grid = n // blk
    return pl.pallas_call(
        _table_body,
        grid=(grid,),
        in_specs=[
            pl.BlockSpec((blk, D), lambda i: (i, 0)),
            pl.BlockSpec((D, DC), lambda i: (0, 0)),
            pl.BlockSpec((1, DC), lambda i: (0, 0)),
        ],
        out_specs=pl.BlockSpec((blk, DC), lambda i: (i, 0)),
        out_shape=jax.ShapeDtypeStruct((n, DC), jnp.bfloat16),
    )(x, wcat, bcat)


def _make_edge_kernel(n_edges):
    per_w = n_edges // NW
    n_chunks = per_w // CH

    mesh = plsc.VectorSubcoreMesh(core_axis_name="c", subcore_axis_name="s")

    @functools.partial(
        pl.kernel,
        mesh=mesh,
        out_type=jax.ShapeDtypeStruct((n_edges, D), jnp.float32),
        scratch_types=[
            pltpu.VMEM((n_chunks, CH), jnp.int32),
            pltpu.VMEM((n_chunks, CH), jnp.int32),
        ]
        + [pltpu.VMEM((CH, D), jnp.int32) for _ in range(2 * NBUF)]
        + [pltpu.VMEM((CH, D), jnp.float32) for _ in range(NBUF)]
        + [pltpu.SemaphoreType.DMA for _ in range(2 * NBUF)],
    )
    def edge_kernel(c_hbm, src_hbm, dst_hbm, out_hbm,
                    sidx, didx, srow0, srow1, drow0, drow1, orow0, orow1,
                    gsem0, gsem1, wsem0, wsem1):
        srow = (srow0, srow1)
        drow = (drow0, drow1)
        orow = (orow0, orow1)
        gsem = (gsem0, gsem1)
        wsem = (wsem0, wsem1)
        wid = lax.axis_index("s") * NC + lax.axis_index("c")
        base = wid * per_w

        pltpu.sync_copy(src_hbm.at[wid], sidx)
        pltpu.sync_copy(dst_hbm.at[wid], didx)

        def fire_gathers(b, ci):
            pltpu.async_copy(c_hbm.at[sidx.at[ci]], srow[b], gsem[b])
            pltpu.async_copy(c_hbm.at[didx.at[ci]], drow[b], gsem[b])

        def wait_gathers(b):
            cp = pltpu.make_async_copy(c_hbm.at[sidx.at[0]], srow[b], gsem[b])
            cp.wait()
            cp = pltpu.make_async_copy(c_hbm.at[didx.at[0]], drow[b], gsem[b])
            cp.wait()

        def wait_write(b):
            pltpu.make_async_copy(
                orow[b], out_hbm.at[pl.ds(base, CH)], wsem[b]).wait()

        for b in range(NBUF):
            fire_gathers(b, b)

        def pair_body(ci0_half, carry):
            ci0 = ci0_half * NBUF
            for b in range(NBUF):
                ci = ci0 + b
                wait_gathers(b)

                @pl.when(ci0 >= NBUF - b)
                def _():
                    wait_write(b)

                def row_body(i, rcarry):
                    hi_mask = jnp.int32(-65536)
                    for j in range(D // LANES):
                        su = srow[b][i, pl.ds(LANES * j, LANES)]
                        du = drow[b][i, pl.ds(LANES * j, LANES)]
                        sa = lax.bitcast_convert_type(su << 16, jnp.float32)
                        sb = lax.bitcast_convert_type(su & hi_mask, jnp.float32)
                        da = lax.bitcast_convert_type(du << 16, jnp.float32)
                        db = lax.bitcast_convert_type(du & hi_mask, jnp.float32)
                        orow[b][i, pl.ds(LANES * j, LANES)] = (
                            jnp.maximum(sa + db, 0.0)
                            + jnp.maximum(da + sb, 0.0)
                        )
                    return rcarry

                lax.fori_loop(0, CH, row_body, 0)
                pltpu.async_copy(
                    orow[b], out_hbm.at[pl.ds(base + ci * CH, CH)], wsem[b])

                @pl.when(ci0 + NBUF < n_chunks)
                def _():
                    fire_gathers(b, ci + NBUF)

            return carry

        lax.fori_loop(0, n_chunks // NBUF, pair_body, 0)

        for b in range(NBUF):
            wait_write(b)

    return edge_kernel


def kernel(x, edge_index, W, b):
    n_edges = edge_index.shape[1]
    per_w = n_edges // NW
    n_chunks = per_w // CH
    wcat = jnp.concatenate([W[:D], W[D:]], axis=1)
    bcat = jnp.concatenate([jnp.zeros_like(b), b]).reshape(1, DC)
    perm = jnp.stack([jnp.arange(D), jnp.arange(D) + D], axis=1).reshape(-1)
    c_bf = _build_table(x, wcat[:, perm], bcat[:, perm])
    c = lax.bitcast_convert_type(
        c_bf.reshape(x.shape[0], D, 2), jnp.int32)
    src = edge_index[0].astype(jnp.int32).reshape(NW, n_chunks, CH)
    dst = edge_index[1].astype(jnp.int32).reshape(NW, n_chunks, CH)
    return _make_edge_kernel(n_edges)(c, src, dst)
